# baseline (speedup 1.0000x reference)
import jax
import jax.numpy as jnp
from jax import lax
from jax.experimental import pallas as pl
from jax.experimental.pallas import tpu as pltpu
from jax.experimental.pallas import tpu_sc as plsc

B = 16384
HIST = 50
D = 64

NC = 2
NS = 16
NW = NC * NS

U_PER_W = B // NW
CHUNK_U = 4
N_CHUNKS = U_PER_W // CHUNK_U
NBUF = 4
ROWS = CHUNK_U * HIST
IDX_SPLIT = (128, 72)
URING = 8


def _sc_body(urows_hbm, midx_hbm, mtab_hbm, out_hbm,
             midx_v, urows_v, *scratch):
    mrows = scratch[0:NBUF]
    outs = scratch[NBUF:2 * NBUF]
    gsems = scratch[2 * NBUF:3 * NBUF]
    osems = scratch[3 * NBUF:4 * NBUF]
    wid = lax.axis_index("s") * NC + lax.axis_index("c")
    wbase = wid * U_PER_W

    lane = lax.iota(jnp.int32, 16)
    out_mask = (lane == 7) | (lane == 15)
    lo_half = lane < 8
    out_off = jnp.where(lo_half, 0, 1)
    bfly = [lane ^ d for d in (8, 4, 2, 1)]

    def movie_descs(c, b):
        descs = []
        off = 0
        for n in IDX_SPLIT:
            descs.append(pltpu.make_async_copy(
                mtab_hbm.at[midx_v.at[pl.ds(c * ROWS + off, n)]],
                mrows[b].at[pl.ds(off, n)], gsems[b]))
            off += n
        return descs

    def issue(c, b):
        for d in movie_descs(c, b):
            d.start()

    def wait(b):
        for d in movie_descs(0, b):
            d.wait()

    def out_desc(c, b):
        return pltpu.make_async_copy(
            outs[b], out_hbm.at[pl.ds((wbase + c * CHUNK_U) * HIST, ROWS)],
            osems[b])

    def compute(c, b):
        mrows_v = mrows[b]
        out_v = outs[b]

        @pl.when(c >= NBUF)
        def _():
            out_desc(c - NBUF, b).wait()

        def user_body(i, _):
            g = (c * CHUNK_U + i) * D
            u0 = urows_v[pl.ds(g, 16)]
            u1 = urows_v[pl.ds(g + 16, 16)]
            u2 = urows_v[pl.ds(g + 32, 16)]
            u3 = urows_v[pl.ds(g + 48, 16)]
            base_row = i * HIST

            def dot(r):
                return (mrows_v[r, pl.ds(0, 16)] * u0
                        + mrows_v[r, pl.ds(16, 16)] * u1
                        + mrows_v[r, pl.ds(32, 16)] * u2
                        + mrows_v[r, pl.ds(48, 16)] * u3)

            for h in range(0, HIST, 2):
                ra = base_row + h
                pa = dot(ra)
                pb = dot(ra + 1)
                sa = pa + pa.at[bfly[0]].get(mode="promise_in_bounds")
                sb = pb + pb.at[bfly[0]].get(mode="promise_in_bounds")
                s = jnp.where(lo_half, sa, sb)
                for ix in bfly[1:]:
                    s = s + s.at[ix].get(mode="promise_in_bounds")
                plsc.store_scatter(out_v, [out_off + ra], s, mask=out_mask)
            return 0

        lax.fori_loop(0, CHUNK_U, user_body, 0)
        out_desc(c, b).start()

    pltpu.sync_copy(midx_hbm.at[pl.ds(wbase * HIST, U_PER_W * HIST)], midx_v)
    for c in range(NBUF - 1):
        issue(c, c)
    pltpu.sync_copy(urows_hbm.at[pl.ds(wbase * D, U_PER_W * D)], urows_v)

    def ring_body(gq, _):
        c0 = gq * NBUF
        for b in range(NBUF):
            c = c0 + b

            @pl.when(c + NBUF - 1 < N_CHUNKS)
            def _():
                issue(c + NBUF - 1, (b + NBUF - 1) % NBUF)

            wait(b)
            compute(c, b)
        return 0

    lax.fori_loop(0, N_CHUNKS // NBUF, ring_body, 0)

    for k in range(NBUF):
        out_desc(N_CHUNKS - NBUF + k, k % NBUF).wait()


def _user_body(uid_hbm, utabT_hbm, out_hbm, uidx_v, acc_v, *scratch):
    stages = scratch[:URING]
    sems = scratch[URING:]
    wid = lax.axis_index("s") * NC + lax.axis_index("c")
    wbase = wid * U_PER_W
    lane = lax.iota(jnp.int32, 16)

    pltpu.sync_copy(uid_hbm.at[pl.ds(wbase, U_PER_W)],
                    uidx_v.at[pl.ds(0, U_PER_W)])

    def fetch(i, j):
        u = uidx_v[pl.ds(i, 16)][0]
        colg = pl.multiple_of((u >> 7) * 128, 128)
        pltpu.make_async_copy(utabT_hbm.at[:, pl.ds(colg, 128)],
                              stages[j], sems[j]).start()

    for j in range(URING):
        fetch(j, j)

    def outer(o, _):
        for j in range(URING):
            i = o * URING + j
            pltpu.make_async_copy(utabT_hbm.at[:, pl.ds(0, 128)],
                                  stages[j], sems[j]).wait()
            u = uidx_v[pl.ds(i, 16)][0]
            col = jnp.full((16,), u & 127, jnp.int32)
            for c in range(4):
                acc_v[pl.ds(i * D + c * 16, 16)] = plsc.load_gather(
                    stages[j], [c * 16 + lane, col])

            @pl.when(i + URING < U_PER_W)
            def _():
                fetch(i + URING, j)
        return 0

    lax.fori_loop(0, U_PER_W // URING, outer, 0)
    pltpu.sync_copy(acc_v, out_hbm.at[pl.ds(wbase * D, U_PER_W * D)])


@jax.jit
def _run(uid_flat, midx_flat, user_table_t, movie_table):
    mesh = plsc.VectorSubcoreMesh(core_axis_name="c", subcore_axis_name="s")
    ku = pl.kernel(
        _user_body,
        out_type=jax.ShapeDtypeStruct((B * D,), jnp.float32),
        mesh=mesh,
        scratch_types=[
            pltpu.VMEM((U_PER_W + 16,), jnp.int32),
            pltpu.VMEM((U_PER_W * D,), jnp.float32),
        ] + [pltpu.VMEM((D, 128), jnp.float32) for _ in range(URING)]
          + [pltpu.SemaphoreType.DMA for _ in range(URING)],
        compiler_params=pltpu.CompilerParams(needs_layout_passes=False,
                                             use_tc_tiling_on_sc=True),
    )
    urows = ku(uid_flat, user_table_t)

    k = pl.kernel(
        _sc_body,
        out_type=jax.ShapeDtypeStruct((B * HIST,), jnp.float32),
        mesh=mesh,
        scratch_types=[
            pltpu.VMEM((U_PER_W * HIST,), jnp.int32),
            pltpu.VMEM((U_PER_W * D,), jnp.float32),
        ] + [pltpu.VMEM((ROWS, D), jnp.float32) for _ in range(NBUF)]
          + [pltpu.VMEM((ROWS,), jnp.float32) for _ in range(NBUF)]
          + [pltpu.SemaphoreType.DMA for _ in range(2 * NBUF)],
        compiler_params=pltpu.CompilerParams(needs_layout_passes=False,
                                             use_tc_tiling_on_sc=False),
    )
    return k(urows, midx_flat, movie_table)


def kernel(user_id, movie_title, user_table, movie_table):
    uid_flat = user_id.reshape(B)
    midx_flat = movie_title.reshape(B * HIST)
    out = _run(uid_flat, midx_flat, user_table.T, movie_table)
    return out.reshape(B, HIST)


# speedup vs baseline: 1.0409x; 1.0409x over previous
import jax
import jax.numpy as jnp
from jax import lax
from jax.experimental import pallas as pl
from jax.experimental.pallas import tpu as pltpu
from jax.experimental.pallas import tpu_sc as plsc

B = 16384
HIST = 50
D = 64

NC = 2
NS = 16
NW = NC * NS

U_PER_W = B // NW
CHUNK_U = 8
N_CHUNKS = U_PER_W // CHUNK_U
ROWS = CHUNK_U * HIST
IDX_SPLIT = (128, 128, 128, 16)
URING = 8


def _sc_body(urows_hbm, midx_hbm, mtab_hbm, out_hbm,
             midx_v, urows_v,
             mrows0_v, mrows1_v, out0_v, out1_v,
             gsem0, gsem1, osem0, osem1):
    mrows = (mrows0_v, mrows1_v)
    outs = (out0_v, out1_v)
    gsems = (gsem0, gsem1)
    osems = (osem0, osem1)
    wid = lax.axis_index("s") * NC + lax.axis_index("c")
    wbase = wid * U_PER_W

    lane = lax.iota(jnp.int32, 16)
    out_mask = (lane == 7) | (lane == 15)
    lo_half = lane < 8
    out_off = jnp.where(lo_half, 0, 1)
    bfly = [lane ^ d for d in (8, 4, 2, 1)]

    def movie_descs(c, b):
        descs = []
        off = 0
        for n in IDX_SPLIT:
            descs.append(pltpu.make_async_copy(
                mtab_hbm.at[midx_v.at[pl.ds(c * ROWS + off, n)]],
                mrows[b].at[pl.ds(off, n)], gsems[b]))
            off += n
        return descs

    def issue(c, b):
        for d in movie_descs(c, b):
            d.start()

    def wait(b):
        for d in movie_descs(0, b):
            d.wait()

    def out_desc(c, b):
        return pltpu.make_async_copy(
            outs[b], out_hbm.at[pl.ds((wbase + c * CHUNK_U) * HIST, ROWS)],
            osems[b])

    def compute(c, b):
        mrows_v = mrows[b]
        out_v = outs[b]

        @pl.when(c >= 2)
        def _():
            out_desc(c - 2, b).wait()

        def user_body(i, _):
            g = (c * CHUNK_U + i) * D
            u0 = urows_v[pl.ds(g, 16)]
            u1 = urows_v[pl.ds(g + 16, 16)]
            u2 = urows_v[pl.ds(g + 32, 16)]
            u3 = urows_v[pl.ds(g + 48, 16)]
            base_row = i * HIST

            def dot(r):
                return (mrows_v[r, pl.ds(0, 16)] * u0
                        + mrows_v[r, pl.ds(16, 16)] * u1
                        + mrows_v[r, pl.ds(32, 16)] * u2
                        + mrows_v[r, pl.ds(48, 16)] * u3)

            for h in range(0, HIST, 2):
                ra = base_row + h
                pa = dot(ra)
                pb = dot(ra + 1)
                sa = pa + pa.at[bfly[0]].get(mode="promise_in_bounds")
                sb = pb + pb.at[bfly[0]].get(mode="promise_in_bounds")
                s = jnp.where(lo_half, sa, sb)
                for ix in bfly[1:]:
                    s = s + s.at[ix].get(mode="promise_in_bounds")
                plsc.store_scatter(out_v, [out_off + ra], s, mask=out_mask)
            return 0

        lax.fori_loop(0, CHUNK_U, user_body, 0)
        out_desc(c, b).start()

    pltpu.sync_copy(midx_hbm.at[pl.ds(wbase * HIST, U_PER_W * HIST)], midx_v)
    issue(0, 0)
    pltpu.sync_copy(urows_hbm.at[pl.ds(wbase * D, U_PER_W * D)], urows_v)

    def pair_body(gp, _):
        c0 = gp * 2
        issue(c0 + 1, 1)
        wait(0)
        compute(c0, 0)

        @pl.when(c0 + 2 < N_CHUNKS)
        def _():
            issue(c0 + 2, 0)

        wait(1)
        compute(c0 + 1, 1)
        return 0

    lax.fori_loop(0, N_CHUNKS // 2, pair_body, 0)

    out_desc(N_CHUNKS - 2, 0).wait()
    out_desc(N_CHUNKS - 1, 1).wait()


def _user_body(uid_hbm, utabT_hbm, out_hbm, uidx_v, acc_v, *scratch):
    stages = scratch[:URING]
    sems = scratch[URING:]
    wid = lax.axis_index("s") * NC + lax.axis_index("c")
    wbase = wid * U_PER_W
    lane = lax.iota(jnp.int32, 16)

    pltpu.sync_copy(uid_hbm.at[pl.ds(wbase, U_PER_W)],
                    uidx_v.at[pl.ds(0, U_PER_W)])

    def fetch(i, j):
        u = uidx_v[pl.ds(i, 16)][0]
        colg = pl.multiple_of((u >> 7) * 128, 128)
        pltpu.make_async_copy(utabT_hbm.at[:, pl.ds(colg, 128)],
                              stages[j], sems[j]).start()

    for j in range(URING):
        fetch(j, j)

    def outer(o, _):
        for j in range(URING):
            i = o * URING + j
            pltpu.make_async_copy(utabT_hbm.at[:, pl.ds(0, 128)],
                                  stages[j], sems[j]).wait()
            u = uidx_v[pl.ds(i, 16)][0]
            col = jnp.full((16,), u & 127, jnp.int32)
            for c in range(4):
                acc_v[pl.ds(i * D + c * 16, 16)] = plsc.load_gather(
                    stages[j], [c * 16 + lane, col])

            @pl.when(i + URING < U_PER_W)
            def _():
                fetch(i + URING, j)
        return 0

    lax.fori_loop(0, U_PER_W // URING, outer, 0)
    pltpu.sync_copy(acc_v, out_hbm.at[pl.ds(wbase * D, U_PER_W * D)])


@jax.jit
def _run(uid_flat, midx_flat, user_table_t, movie_table):
    mesh = plsc.VectorSubcoreMesh(core_axis_name="c", subcore_axis_name="s")
    ku = pl.kernel(
        _user_body,
        out_type=jax.ShapeDtypeStruct((B * D,), jnp.float32),
        mesh=mesh,
        scratch_types=[
            pltpu.VMEM((U_PER_W + 16,), jnp.int32),
            pltpu.VMEM((U_PER_W * D,), jnp.float32),
        ] + [pltpu.VMEM((D, 128), jnp.float32) for _ in range(URING)]
          + [pltpu.SemaphoreType.DMA for _ in range(URING)],
        compiler_params=pltpu.CompilerParams(needs_layout_passes=False,
                                             use_tc_tiling_on_sc=True),
    )
    urows = ku(uid_flat, user_table_t)

    k = pl.kernel(
        _sc_body,
        out_type=jax.ShapeDtypeStruct((B * HIST,), jnp.float32),
        mesh=mesh,
        scratch_types=[
            pltpu.VMEM((U_PER_W * HIST,), jnp.int32),
            pltpu.VMEM((U_PER_W * D,), jnp.float32),
            pltpu.VMEM((ROWS, D), jnp.float32),
            pltpu.VMEM((ROWS, D), jnp.float32),
            pltpu.VMEM((ROWS,), jnp.float32),
            pltpu.VMEM((ROWS,), jnp.float32),
            pltpu.SemaphoreType.DMA,
            pltpu.SemaphoreType.DMA,
            pltpu.SemaphoreType.DMA,
            pltpu.SemaphoreType.DMA,
        ],
        compiler_params=pltpu.CompilerParams(needs_layout_passes=False,
                                             use_tc_tiling_on_sc=False),
    )
    return k(urows, midx_flat, movie_table)


def kernel(user_id, movie_title, user_table, movie_table):
    uid_flat = user_id.reshape(B)
    midx_flat = movie_title.reshape(B * HIST)
    out = _run(uid_flat, midx_flat, user_table.T, movie_table)
    return out.reshape(B, HIST)
